# baseline (speedup 1.0000x reference)
import jax
import jax.numpy as jnp
from jax.experimental import pallas as pl
from jax.experimental.pallas import tpu as pltpu

_BM = 400


def _pre_kernel(x_ref, w1_ref, s1_ref):
    s1_ref[...] = jnp.dot(x_ref[...], w1_ref[...],
                          preferred_element_type=jnp.float32
                          ).astype(jnp.bfloat16)


def _fused_kernel(adj_a_ref, adj_b_ref, s1_ref, w2_ref,
                  h1_ref, logits_ref, adj_out_ref,
                  hw2_scr):
    s = pl.program_id(0)
    i = pl.program_id(1)
    hm = _BM // 2

    @pl.when(s == 0)
    def _():
        adj_out_ref[...] = adj_a_ref[...]
        h1a = jnp.maximum(
            jnp.dot(adj_a_ref[...].astype(jnp.bfloat16), s1_ref[...],
                    preferred_element_type=jnp.float32), 0.0)
        h1b = jnp.maximum(
            jnp.dot(adj_b_ref[...].astype(jnp.bfloat16), s1_ref[...],
                    preferred_element_type=jnp.float32), 0.0)
        h1_ref[pl.ds(0, hm), :] = h1a
        h1_ref[pl.ds(hm, hm), :] = h1b
        hw2_scr[pl.ds(i * _BM, hm), :] = jnp.dot(
            h1a, w2_ref[...], preferred_element_type=jnp.float32
        ).astype(jnp.bfloat16)
        hw2_scr[pl.ds(i * _BM + hm, hm), :] = jnp.dot(
            h1b, w2_ref[...], preferred_element_type=jnp.float32
        ).astype(jnp.bfloat16)

    @pl.when(s == 1)
    def _():
        adj_out_ref[...] = adj_b_ref[...]

        def _lsm(x2):
            m = jnp.max(x2, axis=1, keepdims=True)
            e = jnp.exp(x2 - m)
            return (x2 - m) - jnp.log(jnp.sum(e, axis=1, keepdims=True))

        x2a = jnp.maximum(
            jnp.dot(adj_a_ref[...].astype(jnp.bfloat16), hw2_scr[...],
                    preferred_element_type=jnp.float32), 0.0)
        x2b = jnp.maximum(
            jnp.dot(adj_b_ref[...].astype(jnp.bfloat16), hw2_scr[...],
                    preferred_element_type=jnp.float32), 0.0)
        logits_ref[pl.ds(0, hm), :] = _lsm(x2a)
        logits_ref[pl.ds(hm, hm), :] = _lsm(x2b)


def kernel(x, adj, W1, W2):
    n, nfeat = x.shape
    nhid = W1.shape[1]
    nclass = W2.shape[1]
    ns = n // _BM

    s1 = pl.pallas_call(
        _pre_kernel,
        out_shape=jax.ShapeDtypeStruct((n, nhid), jnp.bfloat16),
    )(x, W1)

    full = lambda s, i: (0, 0)
    top = lambda s, i: (2 * i, 0)
    bot = lambda s, i: (2 * i + 1, 0)
    halves = lambda s, i: (2 * i + s, 0)
    ph0 = lambda s, i: (jnp.where(s == 0, i, ns - 1), 0)
    ph1 = lambda s, i: (jnp.where(s == 1, i, 0), 0)

    h1, logits, adj_out = pl.pallas_call(
        _fused_kernel,
        grid=(2, ns),
        in_specs=[
            pl.BlockSpec((_BM // 2, n), top),
            pl.BlockSpec((_BM // 2, n), bot),
            pl.BlockSpec((n, nhid), full),
            pl.BlockSpec((nhid, nclass), full),
        ],
        out_specs=[
            pl.BlockSpec((_BM, nhid), ph0),
            pl.BlockSpec((_BM, nclass), ph1),
            pl.BlockSpec((_BM // 2, n), halves),
        ],
        out_shape=[
            jax.ShapeDtypeStruct((n, nhid), jnp.float32),
            jax.ShapeDtypeStruct((n, nclass), jnp.float32),
            jax.ShapeDtypeStruct((n, n), jnp.float32),
        ],
        scratch_shapes=[
            pltpu.VMEM((n, nclass), jnp.bfloat16),
        ],
        compiler_params=pltpu.CompilerParams(
            dimension_semantics=("arbitrary", "arbitrary"),
            vmem_limit_bytes=63 * 1024 * 1024,
        ),
    )(adj, adj, s1, W2)
    return (logits, h1, adj_out)


# speedup vs baseline: 1.0029x; 1.0029x over previous
import jax
import jax.numpy as jnp
from jax.experimental import pallas as pl
from jax.experimental.pallas import tpu as pltpu

_BM = 400


def _pre_kernel(x_ref, w1_ref, s1_ref):
    s1_ref[...] = jnp.dot(x_ref[...], w1_ref[...],
                          preferred_element_type=jnp.float32
                          ).astype(jnp.bfloat16)


def _fused_kernel(adj_ref, s1_ref, w2_ref,
                  h1_ref, logits_ref, adj_out_ref,
                  hw2_scr):
    s = pl.program_id(0)
    i = pl.program_id(1)
    hm = _BM // 2

    @pl.when(s == 0)
    def _():
        adj_out_ref[...] = adj_ref[pl.ds(0, hm), :]
        h1 = jnp.maximum(
            jnp.dot(adj_ref[...].astype(jnp.bfloat16), s1_ref[...],
                    preferred_element_type=jnp.float32), 0.0)
        h1_ref[...] = h1
        hw2_scr[pl.ds(i * _BM, _BM), :] = jnp.dot(
            h1, w2_ref[...], preferred_element_type=jnp.float32
        ).astype(jnp.bfloat16)

    @pl.when(s == 1)
    def _():
        adj_out_ref[...] = adj_ref[pl.ds(hm, hm), :]
        x2 = jnp.maximum(
            jnp.dot(adj_ref[...].astype(jnp.bfloat16), hw2_scr[...],
                    preferred_element_type=jnp.float32), 0.0)
        m = jnp.max(x2, axis=1, keepdims=True)
        e = jnp.exp(x2 - m)
        logits_ref[...] = (x2 - m) - jnp.log(
            jnp.sum(e, axis=1, keepdims=True))


def kernel(x, adj, W1, W2):
    n, nfeat = x.shape
    nhid = W1.shape[1]
    nclass = W2.shape[1]
    ns = n // _BM

    s1 = pl.pallas_call(
        _pre_kernel,
        out_shape=jax.ShapeDtypeStruct((n, nhid), jnp.bfloat16),
    )(x, W1)

    full = lambda s, i: (0, 0)
    every = lambda s, i: (i, 0)
    halves = lambda s, i: (2 * i + s, 0)
    ph0 = lambda s, i: (jnp.where(s == 0, i, ns - 1), 0)
    ph1 = lambda s, i: (jnp.where(s == 1, i, 0), 0)

    h1, logits, adj_out = pl.pallas_call(
        _fused_kernel,
        grid=(2, ns),
        in_specs=[
            pl.BlockSpec((_BM, n), every),
            pl.BlockSpec((n, nhid), full),
            pl.BlockSpec((nhid, nclass), full),
        ],
        out_specs=[
            pl.BlockSpec((_BM, nhid), ph0),
            pl.BlockSpec((_BM, nclass), ph1),
            pl.BlockSpec((_BM // 2, n), halves),
        ],
        out_shape=[
            jax.ShapeDtypeStruct((n, nhid), jnp.float32),
            jax.ShapeDtypeStruct((n, nclass), jnp.float32),
            jax.ShapeDtypeStruct((n, n), jnp.float32),
        ],
        scratch_shapes=[
            pltpu.VMEM((n, nclass), jnp.bfloat16),
        ],
        compiler_params=pltpu.CompilerParams(
            dimension_semantics=("arbitrary", "arbitrary"),
            vmem_limit_bytes=63 * 1024 * 1024,
        ),
    )(adj, s1, W2)
    return (logits, h1, adj_out)


# speedup vs baseline: 1.0135x; 1.0106x over previous
import jax
import jax.numpy as jnp
from jax.experimental import pallas as pl
from jax.experimental.pallas import tpu as pltpu

_BM = 400


def _fused_kernel(x_ref, adj_ref, w1_ref, w2_ref,
                  h1_ref, logits_ref, adj_out_ref,
                  s1_scr, hw2_scr):
    s = pl.program_id(0)
    i = pl.program_id(1)
    hm = _BM // 2

    @pl.when((s == 0) & (i == 0))
    def _():
        s1_scr[...] = jnp.dot(x_ref[...], w1_ref[...],
                              preferred_element_type=jnp.float32
                              ).astype(jnp.bfloat16)

    @pl.when(s == 0)
    def _():
        adj_out_ref[...] = adj_ref[pl.ds(0, hm), :]
        h1 = jnp.maximum(
            jnp.dot(adj_ref[...].astype(jnp.bfloat16), s1_scr[...],
                    preferred_element_type=jnp.float32), 0.0)
        h1_ref[...] = h1
        hw2_scr[pl.ds(i * _BM, _BM), :] = jnp.dot(
            h1, w2_ref[...], preferred_element_type=jnp.float32
        ).astype(jnp.bfloat16)

    @pl.when(s == 1)
    def _():
        adj_out_ref[...] = adj_ref[pl.ds(hm, hm), :]
        x2 = jnp.maximum(
            jnp.dot(adj_ref[...].astype(jnp.bfloat16), hw2_scr[...],
                    preferred_element_type=jnp.float32), 0.0)
        m = jnp.max(x2, axis=1, keepdims=True)
        e = jnp.exp(x2 - m)
        logits_ref[...] = (x2 - m) - jnp.log(
            jnp.sum(e, axis=1, keepdims=True))


def kernel(x, adj, W1, W2):
    n, nfeat = x.shape
    nhid = W1.shape[1]
    nclass = W2.shape[1]
    ns = n // _BM

    full = lambda s, i: (0, 0)
    every = lambda s, i: (i, 0)
    halves = lambda s, i: (2 * i + s, 0)
    ph0 = lambda s, i: (jnp.where(s == 0, i, ns - 1), 0)
    ph1 = lambda s, i: (jnp.where(s == 1, i, 0), 0)

    h1, logits, adj_out = pl.pallas_call(
        _fused_kernel,
        grid=(2, ns),
        in_specs=[
            pl.BlockSpec((n, nfeat), full),
            pl.BlockSpec((_BM, n), every),
            pl.BlockSpec((nfeat, nhid), full),
            pl.BlockSpec((nhid, nclass), full),
        ],
        out_specs=[
            pl.BlockSpec((_BM, nhid), ph0),
            pl.BlockSpec((_BM, nclass), ph1),
            pl.BlockSpec((_BM // 2, n), halves),
        ],
        out_shape=[
            jax.ShapeDtypeStruct((n, nhid), jnp.float32),
            jax.ShapeDtypeStruct((n, nclass), jnp.float32),
            jax.ShapeDtypeStruct((n, n), jnp.float32),
        ],
        scratch_shapes=[
            pltpu.VMEM((n, nhid), jnp.bfloat16),
            pltpu.VMEM((n, nclass), jnp.bfloat16),
        ],
        compiler_params=pltpu.CompilerParams(
            dimension_semantics=("arbitrary", "arbitrary"),
            vmem_limit_bytes=63 * 1024 * 1024,
        ),
    )(x, adj, W1, W2)
    return (logits, h1, adj_out)
